# Initial kernel scaffold; baseline (speedup 1.0000x reference)
import jax
import jax.numpy as jnp
from jax.experimental import pallas as pl


def kernel(x, edge_index, W1, b1, W_mu, b_mu, W_logstd, b_logstd):
    raise NotImplementedError("write your pallas kernel here")


# speedup vs baseline: 7.0433x; 7.0433x over previous
import functools

import jax
import jax.numpy as jnp
from jax import lax
from jax.experimental import pallas as pl
from jax.experimental.pallas import tpu as pltpu
from jax.experimental.pallas import tpu_sc as plsc

N = 10000
E = 320000
D = 128
NC = 2
NS = 16
NW = NC * NS
K = 128
NCHUNK = 80
EP = NW * NCHUNK * K
NP = 10240
RPS = NP // NS

_mesh = plsc.VectorSubcoreMesh(core_axis_name="c", subcore_axis_name="s")


@functools.partial(
    pl.kernel,
    out_type=jax.ShapeDtypeStruct((NC * NP,), jnp.float32),
    mesh=_mesh,
    scratch_types=[
        pltpu.VMEM((NCHUNK, K), jnp.int32),
        pltpu.VMEM((K,), jnp.float32),
        pltpu.VMEM((RPS,), jnp.float32),
        pltpu.VMEM_SHARED((NP,), jnp.float32),
    ],
)
def _deg_count(dst_hbm, cnt_hbm, dst_v, ones_v, zeros_v, acc):
    cid = lax.axis_index("c")
    sid = lax.axis_index("s")
    wid = sid * NC + cid

    def fill(j, carry):
        ones_v[pl.ds(j * 16, 16)] = jnp.full((16,), 1.0, jnp.float32)
        return carry

    lax.fori_loop(0, K // 16, fill, 0)

    def zfill(j, carry):
        zeros_v[pl.ds(j * 16, 16)] = jnp.zeros((16,), jnp.float32)
        return carry

    lax.fori_loop(0, RPS // 16, zfill, 0)
    pltpu.sync_copy(zeros_v, acc.at[pl.ds(sid * RPS, RPS)])
    pltpu.sync_copy(dst_hbm.at[wid], dst_v)
    plsc.subcore_barrier()

    def chunk(j, carry):
        pltpu.sync_copy(ones_v, acc.at[dst_v.at[j]], add=True)
        return carry

    lax.fori_loop(0, NCHUNK, chunk, 0)
    plsc.subcore_barrier()
    pltpu.sync_copy(acc.at[pl.ds(sid * RPS, RPS)],
                    cnt_hbm.at[pl.ds(cid * NP + sid * RPS, RPS)])


def _make_agg(n_half):
    @functools.partial(
        pl.kernel,
        out_type=jax.ShapeDtypeStruct((NC * n_half * NP, D), jnp.float32),
        mesh=_mesh,
        scratch_types=[
            pltpu.VMEM((NCHUNK, K), jnp.int32),
            pltpu.VMEM((NCHUNK, K), jnp.int32),
            pltpu.VMEM((K, D), jnp.float32),
            pltpu.VMEM_SHARED((NP, D), jnp.float32),
            pltpu.SemaphoreType.DMA,
        ],
    )
    def _agg(u_hbm, src_hbm, dst_hbm, zeros_hbm, out_hbm,
             src_v, dst_v, rows_v, acc, sem):
        cid = lax.axis_index("c")
        sid = lax.axis_index("s")
        wid = sid * NC + cid
        pltpu.sync_copy(dst_hbm.at[wid], dst_v)
        for h in range(n_half):
            pltpu.sync_copy(zeros_hbm.at[pl.ds(sid * RPS, RPS)],
                            acc.at[pl.ds(sid * RPS, RPS)])
            pltpu.sync_copy(src_hbm.at[h, wid], src_v)
            plsc.subcore_barrier()

            def chunk(j, carry):
                pltpu.async_copy(u_hbm.at[src_v.at[j]], rows_v, sem).wait()
                pltpu.sync_copy(rows_v, acc.at[dst_v.at[j]], add=True)
                return carry

            lax.fori_loop(0, NCHUNK, chunk, 0)
            plsc.subcore_barrier()
            out_row = (cid * n_half + h) * NP + sid * RPS
            pltpu.sync_copy(acc.at[pl.ds(sid * RPS, RPS)],
                            out_hbm.at[pl.ds(out_row, RPS)])

    return _agg


_agg1 = _make_agg(1)
_agg2 = _make_agg(2)


def _tc_a_body(cnt_ref, x_ref, dinv_ref, u1_ref):
    deg = cnt_ref[:, 0:1] + cnt_ref[:, 1:2] + 1.0
    dinv = lax.rsqrt(deg)
    dinv_ref[...] = dinv
    u1_ref[...] = x_ref[...] * dinv


_tc_a = pl.pallas_call(
    _tc_a_body,
    out_shape=(jax.ShapeDtypeStruct((N, 1), jnp.float32),
               jax.ShapeDtypeStruct((N, D), jnp.float32)),
)


def _tc_b_body(s0_ref, s1_ref, u1_ref, dinv_ref, w1_ref, b1_ref,
               u2a_ref, u2b_ref):
    dinv = dinv_ref[...]
    agg = (s0_ref[...] + s1_ref[...] + u1_ref[...]) * dinv
    h = jnp.dot(agg, w1_ref[...], preferred_element_type=jnp.float32)
    h = jnp.maximum(h + b1_ref[...], 0.0)
    u2 = h * dinv
    u2a_ref[...] = u2[:, :D]
    u2b_ref[...] = u2[:, D:]


_tc_b = pl.pallas_call(
    _tc_b_body,
    out_shape=(jax.ShapeDtypeStruct((N, D), jnp.float32),
               jax.ShapeDtypeStruct((N, D), jnp.float32)),
)


def _tc_c_body(p0a_ref, p0b_ref, p1a_ref, p1b_ref, u2a_ref, u2b_ref,
               dinv_ref, wm_ref, bm_ref, wl_ref, bl_ref, mu_ref, ls_ref):
    dinv = dinv_ref[...]
    aa = (p0a_ref[...] + p1a_ref[...] + u2a_ref[...]) * dinv
    ab = (p0b_ref[...] + p1b_ref[...] + u2b_ref[...]) * dinv
    a = jnp.concatenate([aa, ab], axis=1)
    mu_ref[...] = jnp.dot(a, wm_ref[...],
                          preferred_element_type=jnp.float32) + bm_ref[...]
    ls_ref[...] = jnp.dot(a, wl_ref[...],
                          preferred_element_type=jnp.float32) + bl_ref[...]


_tc_c = pl.pallas_call(
    _tc_c_body,
    out_shape=(jax.ShapeDtypeStruct((N, D), jnp.float32),
               jax.ShapeDtypeStruct((N, D), jnp.float32)),
)


def _pad(a):
    return jnp.concatenate([a, jnp.zeros((NP - N, a.shape[1]), a.dtype)], axis=0)


def kernel(x, edge_index, W1, b1, W_mu, b_mu, W_logstd, b_logstd):
    ei = edge_index.astype(jnp.int32)
    src = jnp.concatenate([ei[0], jnp.zeros((EP - E,), jnp.int32)])
    dst = jnp.concatenate([ei[1], jnp.full((EP - E,), N, jnp.int32)])
    src3 = src.reshape(NW, NCHUNK, K)
    dst3 = dst.reshape(NW, NCHUNK, K)
    src_p1 = src3[None]
    src_p2 = jnp.stack([src3, src3 + NP])
    zerosD = jnp.zeros((NP, D), jnp.float32)

    cnt = _deg_count(dst3)
    cnt2 = cnt.reshape(NC, NP)[:, :N]
    dinv, u1 = _tc_a(cnt2.T, x)
    u1p = _pad(u1)

    s = _agg1(u1p, src_p1, dst3, zerosD).reshape(NC, NP, D)[:, :N]
    u2a, u2b = _tc_b(s[0], s[1], u1, dinv, W1, b1.reshape(1, 2 * D))

    u2rows = jnp.concatenate([_pad(u2a), _pad(u2b)], axis=0)
    p = _agg2(u2rows, src_p2, dst3, zerosD).reshape(NC, 2, NP, D)[:, :, :N]
    mu, logstd = _tc_c(p[0, 0], p[0, 1], p[1, 0], p[1, 1], u2a, u2b, dinv,
                       W_mu, b_mu.reshape(1, D), W_logstd, b_logstd.reshape(1, D))
    return (mu, logstd)
